# baseline (speedup 1.0000x reference)
import functools
import jax
import jax.numpy as jnp
from jax import lax
from jax.experimental import pallas as pl
from jax.experimental.pallas import tpu as pltpu
from jax.experimental.pallas import tpu_sc as plsc

N_ROWS = 100000
DIM = 128
NW = 32
CHUNK = 800
CHUNKS_PER_W = -(-N_ROWS // (CHUNK * NW))
LAST_START = N_ROWS - CHUNK

_mesh = plsc.VectorSubcoreMesh(core_axis_name="c", subcore_axis_name="s")


@functools.partial(
    pl.kernel,
    mesh=_mesh,
    out_type=jax.ShapeDtypeStruct((N_ROWS, DIM), jnp.float32),
    scratch_types=[
        pltpu.VMEM((CHUNK, DIM), jnp.float32),
        pltpu.SemaphoreType.DMA,
    ],
)
def _sc_broadcast(table_hbm, out_hbm, buf, sem):
    wid = lax.axis_index("s") * 2 + lax.axis_index("c")
    pltpu.sync_copy(table_hbm, buf.at[pl.ds(0, 1)])
    row = [buf[0, pl.ds(16 * j, 16)] for j in range(DIM // 16)]

    def _fill(r, carry):
        for u in range(4):
            for j in range(DIM // 16):
                buf[r * 4 + u, pl.ds(16 * j, 16)] = row[j]
        return carry

    lax.fori_loop(1, CHUNK // 4, _fill, 0)
    for u in range(1, 4):
        for j in range(DIM // 16):
            buf[u, pl.ds(16 * j, 16)] = row[j]

    copies = []
    for k in range(CHUNKS_PER_W):
        start = jnp.minimum((wid * CHUNKS_PER_W + k) * CHUNK, LAST_START)
        copies.append(pltpu.async_copy(buf, out_hbm.at[pl.ds(start, CHUNK)], sem))
    for c in copies:
        c.wait()


def kernel(indices, table):
    del indices
    return _sc_broadcast(table)


# speedup vs baseline: 1.0476x; 1.0476x over previous
import functools
import jax
import jax.numpy as jnp
from jax import lax
from jax.experimental import pallas as pl
from jax.experimental.pallas import tpu as pltpu
from jax.experimental.pallas import tpu_sc as plsc

N_ROWS = 100000
DIM = 128
NW = 32
CHUNK = 400
CHUNKS_PER_W = -(-N_ROWS // (CHUNK * NW))
LAST_START = N_ROWS - CHUNK

_mesh = plsc.VectorSubcoreMesh(core_axis_name="c", subcore_axis_name="s")


@functools.partial(
    pl.kernel,
    mesh=_mesh,
    out_type=jax.ShapeDtypeStruct((N_ROWS, DIM), jnp.float32),
    scratch_types=[
        pltpu.VMEM((CHUNK, DIM), jnp.float32),
        pltpu.SemaphoreType.DMA,
    ],
)
def _sc_broadcast(table_hbm, out_hbm, buf, sem):
    wid = lax.axis_index("s") * 2 + lax.axis_index("c")
    pltpu.sync_copy(table_hbm, buf.at[pl.ds(0, 1)])
    row = [buf[0, pl.ds(16 * j, 16)] for j in range(DIM // 16)]

    def _fill(r, carry):
        for u in range(4):
            for j in range(DIM // 16):
                buf[r * 4 + u, pl.ds(16 * j, 16)] = row[j]
        return carry

    lax.fori_loop(1, CHUNK // 4, _fill, 0)
    for u in range(1, 4):
        for j in range(DIM // 16):
            buf[u, pl.ds(16 * j, 16)] = row[j]

    copies = []
    for k in range(CHUNKS_PER_W):
        start = jnp.minimum((wid * CHUNKS_PER_W + k) * CHUNK, LAST_START)
        copies.append(pltpu.async_copy(buf, out_hbm.at[pl.ds(start, CHUNK)], sem))
    for c in copies:
        c.wait()


def kernel(indices, table):
    del indices
    return _sc_broadcast(table)
